# Initial kernel scaffold; baseline (speedup 1.0000x reference)
import jax
import jax.numpy as jnp
from jax.experimental import pallas as pl


def kernel(x):
    raise NotImplementedError("write your pallas kernel here")


# speedup vs baseline: 51.2425x; 51.2425x over previous
import functools
import math

import jax
import jax.numpy as jnp
from jax import lax
from jax.experimental import pallas as pl
from jax.experimental.pallas import tpu as pltpu
from jax.experimental.pallas import tpu_sc as plsc

H = 64
W = 900
FOV_UP = 3.0 / 180.0 * math.pi
FOV_DOWN = -25.0 / 180.0 * math.pi
FOV = FOV_UP - FOV_DOWN
P = H * W

NPART = 4
PART = P // NPART
CHUNK = 4096
LANES = 16


def _tc_project_body(xx_ref, xy_ref, xz_ref, idx_ref, d_ref):
    xx = xx_ref[...]
    xy = xy_ref[...]
    xz = xz_ref[...]
    depth = jnp.sqrt(xx * xx + xy * xy + xz * xz)
    yaw = -jnp.arctan2(xy, xx)
    w = jnp.clip(xz / (depth + 1e-8), -1.0, 1.0)
    pitch = 2.0 * jnp.arctan2(w, 1.0 + jnp.sqrt((1.0 + w) * (1.0 - w)))
    vf = jnp.clip((pitch - FOV_DOWN) / FOV * H, 0.0, float(H - 1))
    uf = jnp.clip(0.5 * (yaw / math.pi + 1.0) * W, 0.0, float(W - 1))
    idx_ref[...] = vf.astype(jnp.int32) * W + uf.astype(jnp.int32)
    d_ref[...] = depth


def _tc_project(xx, xy, xz):
    M, C = xx.shape
    blk = (128, C)
    grid = (M // blk[0],)
    spec = pl.BlockSpec(blk, lambda i: (i, 0))
    return pl.pallas_call(
        _tc_project_body,
        grid=grid,
        in_specs=[spec, spec, spec],
        out_specs=[spec, spec],
        out_shape=[
            jax.ShapeDtypeStruct((M, C), jnp.int32),
            jax.ShapeDtypeStruct((M, C), jnp.float32),
        ],
    )(xx, xy, xz)


def _sc_scatter_body(idx_hbm, xx_hbm, xy_hbm, xz_hbm, d_hbm, out_hbm,
                     im0, im1, im2, im3, im4, ib, xb, yb, zb, db):
    B = idx_hbm.shape[0]
    N = idx_hbm.shape[1]
    wid = lax.axis_index("s") * 2 + lax.axis_index("c")
    b = wid // NPART
    lo = (wid % NPART) * PART
    planes = (im0, im1, im2, im3, im4)

    zeros = jnp.zeros((LANES,), jnp.float32)
    ones = jnp.ones((LANES,), jnp.float32)

    def zero_body(i, _):
        for c in range(5):
            planes[c][pl.ds(i * LANES, LANES)] = zeros
        return 0

    lax.fori_loop(0, PART // LANES, zero_body, 0)

    def chunk_body(j, _):
        off = j * CHUNK
        pltpu.sync_copy(idx_hbm.at[b, pl.ds(off, CHUNK)], ib)
        pltpu.sync_copy(xx_hbm.at[b, pl.ds(off, CHUNK)], xb)
        pltpu.sync_copy(xy_hbm.at[b, pl.ds(off, CHUNK)], yb)
        pltpu.sync_copy(xz_hbm.at[b, pl.ds(off, CHUNK)], zb)
        pltpu.sync_copy(d_hbm.at[b, pl.ds(off, CHUNK)], db)

        def inner(i, _):
            sl = pl.ds(i * LANES, LANES)
            loc = ib[sl] - lo
            msk = (loc >= 0) & (loc < PART)
            locc = jnp.minimum(jnp.maximum(loc, 0), PART - 1)
            plsc.store_scatter(im0, [locc], xb[sl], mask=msk)
            plsc.store_scatter(im1, [locc], yb[sl], mask=msk)
            plsc.store_scatter(im2, [locc], zb[sl], mask=msk)
            plsc.store_scatter(im3, [locc], db[sl], mask=msk)
            plsc.store_scatter(im4, [locc], ones, mask=msk)
            return 0

        lax.fori_loop(0, CHUNK // LANES, inner, 0)
        return 0

    lax.fori_loop(0, N // CHUNK, chunk_body, 0)
    for c in range(5):
        pltpu.sync_copy(planes[c], out_hbm.at[pl.ds(b * (5 * P) + c * P + lo, PART)])


def _sc_scatter(idx, xx, xy, xz, d):
    B, N = idx.shape
    mesh = plsc.VectorSubcoreMesh(
        core_axis_name="c", subcore_axis_name="s", num_cores=2, num_subcores=16
    )
    return pl.kernel(
        _sc_scatter_body,
        out_type=jax.ShapeDtypeStruct((B * 5 * P,), jnp.float32),
        mesh=mesh,
        compiler_params=pltpu.CompilerParams(needs_layout_passes=False),
        scratch_types=[
            pltpu.VMEM((PART,), jnp.float32),
            pltpu.VMEM((PART,), jnp.float32),
            pltpu.VMEM((PART,), jnp.float32),
            pltpu.VMEM((PART,), jnp.float32),
            pltpu.VMEM((PART,), jnp.float32),
            pltpu.VMEM((CHUNK,), jnp.int32),
            pltpu.VMEM((CHUNK,), jnp.float32),
            pltpu.VMEM((CHUNK,), jnp.float32),
            pltpu.VMEM((CHUNK,), jnp.float32),
            pltpu.VMEM((CHUNK,), jnp.float32),
        ],
    )(idx, xx, xy, xz, d)


def kernel(x):
    B, N, _ = x.shape
    xx = x[..., 0]
    xy = x[..., 1]
    xz = x[..., 2]
    M = (B * N) // 1024
    idx, d = _tc_project(
        xx.reshape(M, 1024), xy.reshape(M, 1024), xz.reshape(M, 1024)
    )
    idx = idx.reshape(B, N)
    d = d.reshape(B, N)
    out = _sc_scatter(idx, xx, xy, xz, d)
    return out.reshape(B, 5, H, W)


# speedup vs baseline: 130.3270x; 2.5433x over previous
import functools
import math

import jax
import jax.numpy as jnp
from jax import lax
from jax.experimental import pallas as pl
from jax.experimental.pallas import tpu as pltpu
from jax.experimental.pallas import tpu_sc as plsc

H = 64
W = 900
FOV_UP = 3.0 / 180.0 * math.pi
FOV_DOWN = -25.0 / 180.0 * math.pi
FOV = FOV_UP - FOV_DOWN
P = H * W

CHUNK = 2048
LANES = 16
UNROLL = 8


def _tc_project_body(xx_ref, xy_ref, xz_ref, idx_ref, d_ref):
    xx = xx_ref[...]
    xy = xy_ref[...]
    xz = xz_ref[...]
    depth = jnp.sqrt(xx * xx + xy * xy + xz * xz)
    yaw = -jnp.arctan2(xy, xx)
    w = jnp.clip(xz / (depth + 1e-8), -1.0, 1.0)
    pitch = 2.0 * jnp.arctan2(w, 1.0 + jnp.sqrt((1.0 + w) * (1.0 - w)))
    vf = jnp.clip((pitch - FOV_DOWN) / FOV * H, 0.0, float(H - 1))
    uf = jnp.clip(0.5 * (yaw / math.pi + 1.0) * W, 0.0, float(W - 1))
    idx_ref[...] = vf.astype(jnp.int32) * W + uf.astype(jnp.int32)
    d_ref[...] = depth


def _tc_project(xx, xy, xz):
    M, C = xx.shape
    blk = (128, C)
    grid = (M // blk[0],)
    spec = pl.BlockSpec(blk, lambda i: (i, 0))
    return pl.pallas_call(
        _tc_project_body,
        grid=grid,
        in_specs=[spec, spec, spec],
        out_specs=[spec, spec],
        out_shape=[
            jax.ShapeDtypeStruct((M, C), jnp.int32),
            jax.ShapeDtypeStruct((M, C), jnp.float32),
        ],
    )(xx, xy, xz)


def _sc_scatter_body(idx_hbm, xx_hbm, xy_hbm, xz_hbm, d_hbm, out_hbm,
                     im, im2, iba, ibb, vba, vbb, sema, semb):
    N = idx_hbm.shape[1]
    nch = N // CHUNK
    wid = lax.axis_index("s") * 2 + lax.axis_index("c")
    b = wid // 4
    r = wid % 4

    zeros = jnp.zeros((LANES,), jnp.float32)

    def zero_body(i, _):
        base = i * (LANES * UNROLL)
        for u in range(UNROLL):
            im[pl.ds(base + u * LANES, LANES)] = zeros
        return 0

    lax.fori_loop(0, P // (LANES * UNROLL), zero_body, 0)

    def run_channel(src_hbm):
        def issue(off, ib, vb, sem):
            pltpu.async_copy(idx_hbm.at[b, pl.ds(off, CHUNK)], ib, sem)
            pltpu.async_copy(src_hbm.at[b, pl.ds(off, CHUNK)], vb, sem)

        def drain(ib, vb, sem):
            pltpu.make_async_copy(idx_hbm.at[b, pl.ds(0, CHUNK)], ib, sem).wait()
            pltpu.make_async_copy(src_hbm.at[b, pl.ds(0, CHUNK)], vb, sem).wait()

        def process(ib, vb):
            def inner(i, _):
                base = i * (LANES * UNROLL)
                for u in range(UNROLL):
                    sl = pl.ds(base + u * LANES, LANES)
                    plsc.store_scatter(im, [ib[sl]], vb[sl])
                return 0

            lax.fori_loop(0, CHUNK // (LANES * UNROLL), inner, 0)

        issue(0, iba, vba, sema)

        def body(j2, _):
            offb = (2 * j2 + 1) * CHUNK
            issue(offb, ibb, vbb, semb)
            drain(iba, vba, sema)
            process(iba, vba)
            offa = jnp.minimum((2 * j2 + 2) * CHUNK, N - CHUNK)
            issue(offa, iba, vba, sema)
            drain(ibb, vbb, semb)
            process(ibb, vbb)
            return 0

        lax.fori_loop(0, nch // 2, body, 0)
        drain(iba, vba, sema)

    for rr, src in enumerate((xx_hbm, xy_hbm, xz_hbm, d_hbm)):
        @pl.when(r == rr)
        def _(src=src):
            run_channel(src)

    pltpu.sync_copy(im, out_hbm.at[pl.ds(b * (5 * P) + r * P, P)])

    @pl.when(r == 3)
    def _():
        ones = jnp.ones((LANES,), jnp.float32)

        def mask_body(i, _):
            base = i * (LANES * UNROLL)
            for u in range(UNROLL):
                sl = pl.ds(base + u * LANES, LANES)
                im2[sl] = jnp.where(im[sl] > 0.0, ones, zeros)
            return 0

        lax.fori_loop(0, P // (LANES * UNROLL), mask_body, 0)
        pltpu.sync_copy(im2, out_hbm.at[pl.ds(b * (5 * P) + 4 * P, P)])


def _sc_scatter(idx, xx, xy, xz, d):
    B, N = idx.shape
    mesh = plsc.VectorSubcoreMesh(
        core_axis_name="c", subcore_axis_name="s", num_cores=2, num_subcores=16
    )
    return pl.kernel(
        _sc_scatter_body,
        out_type=jax.ShapeDtypeStruct((B * 5 * P,), jnp.float32),
        mesh=mesh,
        compiler_params=pltpu.CompilerParams(needs_layout_passes=False),
        scratch_types=[
            pltpu.VMEM((P,), jnp.float32),
            pltpu.VMEM((P,), jnp.float32),
            pltpu.VMEM((CHUNK,), jnp.int32),
            pltpu.VMEM((CHUNK,), jnp.int32),
            pltpu.VMEM((CHUNK,), jnp.float32),
            pltpu.VMEM((CHUNK,), jnp.float32),
            pltpu.SemaphoreType.DMA,
            pltpu.SemaphoreType.DMA,
        ],
    )(idx, xx, xy, xz, d)


def kernel(x):
    B, N, _ = x.shape
    xx = x[..., 0]
    xy = x[..., 1]
    xz = x[..., 2]
    M = (B * N) // 1024
    idx, d = _tc_project(
        xx.reshape(M, 1024), xy.reshape(M, 1024), xz.reshape(M, 1024)
    )
    idx = idx.reshape(B, N)
    d = d.reshape(B, N)
    out = _sc_scatter(idx, xx, xy, xz, d)
    return out.reshape(B, 5, H, W)
